# baseline (speedup 1.0000x reference)
import functools

import jax
import jax.numpy as jnp
from jax import lax
from jax.experimental import pallas as pl
from jax.experimental.pallas import tpu as pltpu
from jax.experimental.pallas import tpu_sc as plsc

_EMB = 128
_NC, _NS, _L = 2, 16, 16
_NW = _NC * _NS
_CH = 400
_UNROLL = 16


def _combo_body(w0_ref, w1_ref, w2_ref, out_ref):
    out_ref[...] = jnp.zeros((64, _EMB), jnp.float32)
    for v in range(60):
        a, b, c = v // 12, (v // 2) % 6, v % 2
        out_ref[v : v + 1, :] = (
            w0_ref[a : a + 1, :] + w1_ref[b : b + 1, :] + w2_ref[c : c + 1, :]
        )


def _sc_body(combo_hbm, x0_hbm, x1_hbm, x2_hbm, out_hbm,
             combo_v, i0a, i1a, i2a, i0b, i1b, i2b, ca, cb, ra, rb,
             isems, osems, *, n_chunks, n_iter):
    idx_b = ((i0a, i1a, i2a), (i0b, i1b, i2b))
    cidx_b = (ca, cb)
    rows_b = (ra, rb)
    wid = lax.axis_index("s") * _NC + lax.axis_index("c")

    def chunk_id(m):
        return wid + _NW * m

    def valid(m):
        return chunk_id(m) < n_chunks

    def idx_copies(m, p):
        base = chunk_id(m) * _CH
        return [
            pltpu.make_async_copy(
                x_hbm.at[pl.ds(base, _CH)], dst, isems[p])
            for dst, x_hbm in zip(idx_b[p], (x0_hbm, x1_hbm, x2_hbm))
        ]

    def out_copy(m, p):
        base = chunk_id(m) * _CH * _EMB
        return pltpu.make_async_copy(
            rows_b[p], out_hbm.at[pl.ds(base, _CH * _EMB)], osems[p])

    pltpu.sync_copy(combo_hbm, combo_v)
    iotas = [lax.iota(jnp.int32, _L) + _L * jj for jj in range(_EMB // _L)]

    def half(m, p):
        @pl.when(valid(m))
        def _():
            for cp in idx_copies(m, p):
                cp.wait()
            i0v, i1v, i2v = idx_b[p]
            for grp in range(_CH // _L):
                sl = pl.ds(_L * grp, _L)
                c = (i0v[sl] * 6 + i1v[sl]) * 2 + i2v[sl]
                cidx_b[p][sl] = c * _EMB

            @pl.when(valid(m + 1))
            def _():
                for cp in idx_copies(m + 1, 1 - p):
                    cp.start()

            @pl.when(m >= 2)
            def _():
                out_copy(m - 2, p).wait()

            @plsc.parallel_loop(0, _CH, 1, unroll=_UNROLL)
            def _(e):
                cbase = plsc.load_gather(
                    cidx_b[p], [jnp.full((_L,), 0, jnp.int32) + e])
                for jj in range(_EMB // _L):
                    val = plsc.load_gather(combo_v, [cbase + iotas[jj]])
                    rows_b[p][pl.ds(e * _EMB + _L * jj, _L)] = val

            out_copy(m, p).start()

    @pl.when(valid(0))
    def _():
        for cp in idx_copies(0, 0):
            cp.start()

    def body(t, carry):
        half(2 * t, 0)
        half(2 * t + 1, 1)
        return carry

    lax.fori_loop(0, n_iter, body, 0)

    for p in range(2):
        @pl.when(valid(p))
        def _(p=p):
            out_copy(p, p).wait()


def kernel(inputs, W0, W1, W2):
    E = inputs.shape[0]
    assert E % _CH == 0
    n_chunks = E // _CH
    max_m = -(-n_chunks // _NW)
    n_iter = -(-max_m // 2)
    combo = pl.pallas_call(
        _combo_body,
        out_shape=jax.ShapeDtypeStruct((64, _EMB), jnp.float32),
    )(W0, W1, W2)
    combo_flat = combo.reshape(-1)
    xt = inputs.T
    x0, x1, x2 = xt[0], xt[1], xt[2]

    sc = pl.kernel(
        functools.partial(_sc_body, n_chunks=n_chunks, n_iter=n_iter),
        out_type=jax.ShapeDtypeStruct((E * _EMB,), jnp.float32),
        mesh=plsc.VectorSubcoreMesh(
            core_axis_name="c", subcore_axis_name="s",
            num_cores=_NC, num_subcores=_NS,
        ),
        compiler_params=pltpu.CompilerParams(needs_layout_passes=False),
        scratch_types=[
            pltpu.VMEM((64 * _EMB,), jnp.float32),
            pltpu.VMEM((_CH,), jnp.int32),
            pltpu.VMEM((_CH,), jnp.int32),
            pltpu.VMEM((_CH,), jnp.int32),
            pltpu.VMEM((_CH,), jnp.int32),
            pltpu.VMEM((_CH,), jnp.int32),
            pltpu.VMEM((_CH,), jnp.int32),
            pltpu.VMEM((_CH,), jnp.int32),
            pltpu.VMEM((_CH,), jnp.int32),
            pltpu.VMEM((_CH * _EMB,), jnp.float32),
            pltpu.VMEM((_CH * _EMB,), jnp.float32),
            [pltpu.SemaphoreType.DMA] * 2,
            [pltpu.SemaphoreType.DMA] * 2,
        ],
    )
    out_flat = sc(combo_flat, x0, x1, x2)
    return out_flat.reshape(E, _EMB)


# speedup vs baseline: 1.0189x; 1.0189x over previous
import functools

import jax
import jax.numpy as jnp
from jax import lax
from jax.experimental import pallas as pl
from jax.experimental.pallas import tpu as pltpu
from jax.experimental.pallas import tpu_sc as plsc

_EMB = 128
_NC, _NS, _L = 2, 16, 16
_NW = _NC * _NS
_CH = 400
_UNROLL = 8


def _combo_body(w0_ref, w1_ref, w2_ref, out_ref):
    out_ref[...] = jnp.zeros((64, _EMB), jnp.float32)
    for v in range(60):
        a, b, c = v // 12, (v // 2) % 6, v % 2
        out_ref[v : v + 1, :] = (
            w0_ref[a : a + 1, :] + w1_ref[b : b + 1, :] + w2_ref[c : c + 1, :]
        )


def _sc_body(combo_hbm, x0_hbm, x1_hbm, x2_hbm, out_hbm,
             combo_v, i0a, i1a, i2a, i0b, i1b, i2b, ca, cb, ra, rb,
             isems, osems, *, n_chunks, n_iter):
    idx_b = ((i0a, i1a, i2a), (i0b, i1b, i2b))
    cidx_b = (ca, cb)
    rows_b = (ra, rb)
    wid = lax.axis_index("s") * _NC + lax.axis_index("c")

    def chunk_id(m):
        return wid + _NW * m

    def valid(m):
        return chunk_id(m) < n_chunks

    def idx_copies(m, p):
        base = chunk_id(m) * _CH
        return [
            pltpu.make_async_copy(
                x_hbm.at[pl.ds(base, _CH)], dst, isems[p])
            for dst, x_hbm in zip(idx_b[p], (x0_hbm, x1_hbm, x2_hbm))
        ]

    def out_copy(m, p):
        base = chunk_id(m) * _CH * _EMB
        return pltpu.make_async_copy(
            rows_b[p], out_hbm.at[pl.ds(base, _CH * _EMB)], osems[p])

    pltpu.sync_copy(combo_hbm, combo_v)
    iotas = [lax.iota(jnp.int32, _L) + _L * jj for jj in range(_EMB // _L)]

    def half(m, p):
        @pl.when(valid(m))
        def _():
            for cp in idx_copies(m, p):
                cp.wait()
            i0v, i1v, i2v = idx_b[p]
            for grp in range(_CH // _L):
                sl = pl.ds(_L * grp, _L)
                c = (i0v[sl] * 6 + i1v[sl]) * 2 + i2v[sl]
                cidx_b[p][sl] = c * _EMB

            @pl.when(valid(m + 1))
            def _():
                for cp in idx_copies(m + 1, 1 - p):
                    cp.start()

            @pl.when(m >= 2)
            def _():
                out_copy(m - 2, p).wait()

            @plsc.parallel_loop(0, _CH, 1, unroll=_UNROLL)
            def _(e):
                cbase = plsc.load_gather(
                    cidx_b[p], [jnp.full((_L,), 0, jnp.int32) + e])
                for jj in range(_EMB // _L):
                    val = plsc.load_gather(combo_v, [cbase + iotas[jj]])
                    rows_b[p][pl.ds(e * _EMB + _L * jj, _L)] = val

            out_copy(m, p).start()

    @pl.when(valid(0))
    def _():
        for cp in idx_copies(0, 0):
            cp.start()

    def body(t, carry):
        half(2 * t, 0)
        half(2 * t + 1, 1)
        return carry

    lax.fori_loop(0, n_iter, body, 0)

    for p in range(2):
        @pl.when(valid(p))
        def _(p=p):
            out_copy(p, p).wait()


def kernel(inputs, W0, W1, W2):
    E = inputs.shape[0]
    assert E % _CH == 0
    n_chunks = E // _CH
    max_m = -(-n_chunks // _NW)
    n_iter = -(-max_m // 2)
    combo = pl.pallas_call(
        _combo_body,
        out_shape=jax.ShapeDtypeStruct((64, _EMB), jnp.float32),
    )(W0, W1, W2)
    combo_flat = combo.reshape(-1)
    xt = inputs.T
    x0, x1, x2 = xt[0], xt[1], xt[2]

    sc = pl.kernel(
        functools.partial(_sc_body, n_chunks=n_chunks, n_iter=n_iter),
        out_type=jax.ShapeDtypeStruct((E * _EMB,), jnp.float32),
        mesh=plsc.VectorSubcoreMesh(
            core_axis_name="c", subcore_axis_name="s",
            num_cores=_NC, num_subcores=_NS,
        ),
        compiler_params=pltpu.CompilerParams(needs_layout_passes=False),
        scratch_types=[
            pltpu.VMEM((64 * _EMB,), jnp.float32),
            pltpu.VMEM((_CH,), jnp.int32),
            pltpu.VMEM((_CH,), jnp.int32),
            pltpu.VMEM((_CH,), jnp.int32),
            pltpu.VMEM((_CH,), jnp.int32),
            pltpu.VMEM((_CH,), jnp.int32),
            pltpu.VMEM((_CH,), jnp.int32),
            pltpu.VMEM((_CH,), jnp.int32),
            pltpu.VMEM((_CH,), jnp.int32),
            pltpu.VMEM((_CH * _EMB,), jnp.float32),
            pltpu.VMEM((_CH * _EMB,), jnp.float32),
            [pltpu.SemaphoreType.DMA] * 2,
            [pltpu.SemaphoreType.DMA] * 2,
        ],
    )
    out_flat = sc(combo_flat, x0, x1, x2)
    return out_flat.reshape(E, _EMB)
